# baseline (speedup 1.0000x reference)
import jax
import jax.numpy as jnp
from jax.experimental import pallas as pl
from jax.experimental.pallas import tpu as pltpu

_TEMP_MIN = 2e-05
_NEG_INF = float("-inf")
_W = 1024
_LOG2E = 1.4426950408889634
_LN2 = 0.6931471805599453


def _body(t_ref, minp_ref, eps_ref, eta_ref, x_ref, out_ref, samp_ref,
          sm_ref):
    R, V = x_ref.shape
    NF = V // _W
    TW = V - NF * _W
    toff = NF * _W

    rt2 = _LOG2E / jnp.maximum(t_ref[...], _TEMP_MIN)
    lminp = jnp.log2(minp_ref[...])

    accs = []
    for i in range(NF):
        c = x_ref[:, pl.ds(i * _W, _W)]
        if i < 4:
            accs.append(c)
        else:
            accs[i % 4] = jnp.maximum(accs[i % 4], c)
    acc = jnp.maximum(jnp.maximum(accs[0], accs[1]),
                      jnp.maximum(accs[2], accs[3]))
    xmax = jnp.max(acc, axis=-1, keepdims=True)
    xmax = jnp.maximum(xmax, jnp.max(x_ref[:, pl.ds(toff, TW)],
                                     axis=-1, keepdims=True))

    def p2_chunk(off, w):
        x = x_ref[:, pl.ds(off, w)]
        sm2 = (x - xmax) * rt2
        e = jnp.exp2(sm2)
        sm_ref[:, pl.ds(off, w)] = sm2
        z2c = jnp.where(sm2 >= lminp, e, 0.0)
        idx = jax.lax.broadcasted_iota(jnp.int32, (R, w), 1) + off
        topc = jnp.where(e == 1.0, idx, V)
        return z2c, topc

    z2s, tops = [], []
    for i in range(NF):
        z2c, topc = p2_chunk(i * _W, _W)
        if i < 4:
            z2s.append(z2c)
            tops.append(topc)
        else:
            z2s[i % 4] = z2s[i % 4] + z2c
            tops[i % 4] = jnp.minimum(tops[i % 4], topc)
    z2tc, toptc = p2_chunk(toff, TW)
    z2a = (z2s[0] + z2s[1]) + (z2s[2] + z2s[3])
    topa = jnp.minimum(jnp.minimum(tops[0], tops[1]),
                       jnp.minimum(tops[2], tops[3]))
    z2 = (jnp.sum(z2a, axis=-1, keepdims=True)
          + jnp.sum(z2tc, axis=-1, keepdims=True))
    top_idx = jnp.minimum(jnp.min(topa, axis=-1, keepdims=True),
                          jnp.min(toptc, axis=-1, keepdims=True))

    lthr2 = jnp.maximum(lminp, jnp.log2(eps_ref[...] * z2))

    def p3_chunk(off, w):
        sm2 = sm_ref[:, pl.ds(off, w)]
        z3c = jnp.where(sm2 >= lthr2, jnp.exp2(sm2), 0.0)
        return z3c, z3c * sm2

    z3s, u3s = [], []
    for i in range(NF):
        z3c, u3c = p3_chunk(i * _W, _W)
        if i < 4:
            z3s.append(z3c)
            u3s.append(u3c)
        else:
            z3s[i % 4] = z3s[i % 4] + z3c
            u3s[i % 4] = u3s[i % 4] + u3c
    z3tc, u3tc = p3_chunk(toff, TW)
    z3a = (z3s[0] + z3s[1]) + (z3s[2] + z3s[3])
    u3a = (u3s[0] + u3s[1]) + (u3s[2] + u3s[3])
    z3 = (jnp.sum(z3a, axis=-1, keepdims=True)
          + jnp.sum(z3tc, axis=-1, keepdims=True))
    u3 = (jnp.sum(u3a, axis=-1, keepdims=True)
          + jnp.sum(u3tc, axis=-1, keepdims=True)) * _LN2
    z3 = z3 + jnp.where(lthr2 <= 0.0, 0.0, 1.0)

    neg_ent = u3 / z3 - jnp.log(z3)
    eta = eta_ref[...]
    eps_eta = jnp.minimum(eta, jnp.sqrt(eta) * jnp.exp(neg_ent))
    lthr3 = jnp.maximum(lthr2, jnp.log2(eps_eta * z3))

    def p4_chunk(off, w):
        sm2 = sm_ref[:, pl.ds(off, w)]
        return jnp.where(sm2 >= lthr3, jnp.exp2(sm2), 0.0)

    z4s = []
    for i in range(NF):
        z4c = p4_chunk(i * _W, _W)
        if i < 4:
            z4s.append(z4c)
        else:
            z4s[i % 4] = z4s[i % 4] + z4c
    z4a = (z4s[0] + z4s[1]) + (z4s[2] + z4s[3])
    z4 = (jnp.sum(z4a, axis=-1, keepdims=True)
          + jnp.sum(p4_chunk(toff, TW), axis=-1, keepdims=True))
    z4 = z4 + jnp.where(lthr3 <= 0.0, 0.0, 1.0)
    lz4_2 = jnp.log2(z4)

    lthr3c = jnp.minimum(lthr3, 0.0)

    def p5_chunk(off, w):
        sm2 = sm_ref[:, pl.ds(off, w)]
        out_ref[:, pl.ds(off, w)] = jnp.where(
            sm2 >= lthr3c, (sm2 - lz4_2) * _LN2, _NEG_INF)

    for i in range(NF):
        p5_chunk(i * _W, _W)
    p5_chunk(toff, TW)

    samp_ref[...] = top_idx


def kernel(logits, temperature, min_p, epsilon_cutoff, eta_cutoff):
    B, V = logits.shape
    R = 8
    grid = (B // R,)
    row_spec = pl.BlockSpec((R, 1), lambda i: (i, 0))
    out = pl.pallas_call(
        _body,
        grid=grid,
        in_specs=[row_spec, row_spec, row_spec, row_spec,
                  pl.BlockSpec((R, V), lambda i: (i, 0))],
        out_specs=[pl.BlockSpec((R, V), lambda i: (i, 0)),
                   pl.BlockSpec((R, 1), lambda i: (i, 0))],
        out_shape=[jax.ShapeDtypeStruct((B, V), jnp.float32),
                   jax.ShapeDtypeStruct((B, 1), jnp.int32)],
        scratch_shapes=[pltpu.VMEM((R, V), jnp.float32)],
        compiler_params=pltpu.CompilerParams(
            dimension_semantics=("parallel",)),
    )(temperature.reshape(B, 1), min_p.reshape(B, 1),
      epsilon_cutoff.reshape(B, 1), eta_cutoff.reshape(B, 1), logits)
    return out[0], out[1].reshape(B)


# speedup vs baseline: 1.0461x; 1.0461x over previous
import functools

import jax
import jax.numpy as jnp
from jax.experimental import pallas as pl
from jax.experimental.pallas import tpu as pltpu

_TEMP_MIN = 2e-05
_NEG_INF = float("-inf")
_W = 2048


def _body(t_ref, minp_ref, eps_ref, eta_ref, x_ref, out_ref, samp_ref,
          sm_ref, e_ref):
    R, V = x_ref.shape
    NF = V // _W
    TW = V - NF * _W
    toff = NF * _W

    rt = 1.0 / jnp.maximum(t_ref[...], _TEMP_MIN)
    lminp = jnp.log(minp_ref[...])

    acc = x_ref[:, pl.ds(0, _W)]
    for i in range(1, NF):
        acc = jnp.maximum(acc, x_ref[:, pl.ds(i * _W, _W)])
    xmax = jnp.max(acc, axis=-1, keepdims=True)
    xmax = jnp.maximum(xmax, jnp.max(x_ref[:, pl.ds(toff, TW)],
                                     axis=-1, keepdims=True))

    def p2_chunk(off, w):
        x = x_ref[:, pl.ds(off, w)]
        sm = (x - xmax) * rt
        e = jnp.exp(sm)
        sm_ref[:, pl.ds(off, w)] = sm
        e_ref[:, pl.ds(off, w)] = e
        z2c = jnp.where(sm >= lminp, e, 0.0)
        idx = jax.lax.broadcasted_iota(jnp.int32, (R, w), 1) + off
        topc = jnp.where(e == 1.0, idx, V)
        return z2c, topc

    z2a, topa = p2_chunk(0, _W)
    for i in range(1, NF):
        z2c, topc = p2_chunk(i * _W, _W)
        z2a = z2a + z2c
        topa = jnp.minimum(topa, topc)
    z2tc, toptc = p2_chunk(toff, TW)
    z2 = (jnp.sum(z2a, axis=-1, keepdims=True)
          + jnp.sum(z2tc, axis=-1, keepdims=True))
    top_idx = jnp.minimum(jnp.min(topa, axis=-1, keepdims=True),
                          jnp.min(toptc, axis=-1, keepdims=True))

    lthr2 = jnp.maximum(lminp, jnp.log(eps_ref[...] * z2))

    def p3_chunk(off, w):
        sm = sm_ref[:, pl.ds(off, w)]
        e = e_ref[:, pl.ds(off, w)]
        z3c = jnp.where(sm >= lthr2, e, 0.0)
        return z3c, z3c * sm

    z3a, u3a = p3_chunk(0, _W)
    for i in range(1, NF):
        z3c, u3c = p3_chunk(i * _W, _W)
        z3a = z3a + z3c
        u3a = u3a + u3c
    z3tc, u3tc = p3_chunk(toff, TW)
    z3 = (jnp.sum(z3a, axis=-1, keepdims=True)
          + jnp.sum(z3tc, axis=-1, keepdims=True))
    u3 = (jnp.sum(u3a, axis=-1, keepdims=True)
          + jnp.sum(u3tc, axis=-1, keepdims=True))
    z3 = z3 + jnp.where(lthr2 <= 0.0, 0.0, 1.0)

    neg_ent = u3 / z3 - jnp.log(z3)
    eta = eta_ref[...]
    eps_eta = jnp.minimum(eta, jnp.sqrt(eta) * jnp.exp(neg_ent))
    lthr3 = jnp.maximum(lthr2, jnp.log(eps_eta * z3))

    def p4_chunk(off, w):
        sm = sm_ref[:, pl.ds(off, w)]
        e = e_ref[:, pl.ds(off, w)]
        return jnp.where(sm >= lthr3, e, 0.0)

    z4a = p4_chunk(0, _W)
    for i in range(1, NF):
        z4a = z4a + p4_chunk(i * _W, _W)
    z4 = (jnp.sum(z4a, axis=-1, keepdims=True)
          + jnp.sum(p4_chunk(toff, TW), axis=-1, keepdims=True))
    z4 = z4 + jnp.where(lthr3 <= 0.0, 0.0, 1.0)
    lz4 = jnp.log(z4)

    lthr3c = jnp.minimum(lthr3, 0.0)

    def p5_chunk(off, w):
        sm = sm_ref[:, pl.ds(off, w)]
        out_ref[:, pl.ds(off, w)] = jnp.where(sm >= lthr3c, sm - lz4, _NEG_INF)

    for i in range(NF):
        p5_chunk(i * _W, _W)
    p5_chunk(toff, TW)

    samp_ref[...] = top_idx


def kernel(logits, temperature, min_p, epsilon_cutoff, eta_cutoff):
    B, V = logits.shape
    R = 8
    grid = (B // R,)
    row_spec = pl.BlockSpec((R, 1), lambda i: (i, 0))
    out = pl.pallas_call(
        _body,
        grid=grid,
        in_specs=[row_spec, row_spec, row_spec, row_spec,
                  pl.BlockSpec((R, V), lambda i: (i, 0))],
        out_specs=[pl.BlockSpec((R, V), lambda i: (i, 0)),
                   pl.BlockSpec((R, 1), lambda i: (i, 0))],
        out_shape=[jax.ShapeDtypeStruct((B, V), jnp.float32),
                   jax.ShapeDtypeStruct((B, 1), jnp.int32)],
        scratch_shapes=[pltpu.VMEM((R, V), jnp.float32),
                        pltpu.VMEM((R, V), jnp.float32)],
    )(temperature.reshape(B, 1), min_p.reshape(B, 1),
      epsilon_cutoff.reshape(B, 1), eta_cutoff.reshape(B, 1), logits)
    return out[0], out[1].reshape(B)
